# baseline (speedup 1.0000x reference)
import functools

import jax
import jax.numpy as jnp
from jax import lax
from jax.experimental import pallas as pl
from jax.experimental.pallas import tpu as pltpu
from jax.experimental.pallas import tpu_sc as plsc

MODEL_DIM = 64
SCALE = float(MODEL_DIM) ** 0.5

_info = plsc.get_sparse_core_info()
NC, NS, L = _info.num_cores, _info.num_subcores, _info.num_lanes
NW = NC * NS

CHUNK = 128
NBUF = 5
LEAD = 2

TBLOCK = 2048


def _transpose_scale(table_t):
    d, v = table_t.shape
    grid = pl.cdiv(v, TBLOCK)

    def body(in_ref, out_ref):
        x = in_ref[...]
        row = lax.broadcasted_iota(jnp.int32, (d, 2 * d), 0)
        col = lax.broadcasted_iota(jnp.int32, (d, 2 * d), 1)
        p = jnp.where(row == col, jnp.float32(SCALE), jnp.float32(0.0))
        out_ref[...] = lax.dot_general(
            x, p, (((0,), (0,)), ((), ())),
            preferred_element_type=jnp.float32,
            precision=lax.Precision.HIGHEST,
        )

    return pl.pallas_call(
        body,
        grid=(grid,),
        in_specs=[pl.BlockSpec((d, TBLOCK), lambda g: (0, g))],
        out_specs=pl.BlockSpec((TBLOCK, 2 * d), lambda g: (g, 0)),
        out_shape=jax.ShapeDtypeStruct((v, 2 * d), jnp.float32),
    )(table_t)


def _make_lookup(n_chunks):
    assert n_chunks % NBUF == 0 and n_chunks >= NBUF + LEAD
    n_groups = n_chunks // NBUF
    mesh = plsc.VectorSubcoreMesh(core_axis_name="c", subcore_axis_name="s")

    scratch = [pltpu.VMEM((n_chunks, CHUNK), jnp.int32)]
    scratch += [pltpu.VMEM((CHUNK, MODEL_DIM), jnp.float32) for _ in range(NBUF)]
    scratch += [pltpu.SemaphoreType.DMA for _ in range(2 * NBUF)]

    @functools.partial(
        pl.kernel,
        mesh=mesh,
        compiler_params=pltpu.CompilerParams(use_tc_tiling_on_sc=False),
        out_type=jax.ShapeDtypeStruct((NW, n_chunks, CHUNK, MODEL_DIM), jnp.float32),
        scratch_types=scratch,
    )
    def lookup(idx_hbm, table_hbm, out_hbm, idx_v, *bufs_and_sems):
        bufs = bufs_and_sems[:NBUF]
        gsem = bufs_and_sems[NBUF:2 * NBUF]
        ssem = bufs_and_sems[2 * NBUF:]
        wid = lax.axis_index("s") * NC + lax.axis_index("c")
        pltpu.sync_copy(idx_hbm.at[wid], idx_v)

        for c0 in range(LEAD):
            pltpu.async_copy(table_hbm.at[idx_v.at[c0]], bufs[c0], gsem[c0])

        def group(g, carry):
            for b in range(NBUF):
                c = g * NBUF + b
                r = c + LEAD
                rb = (b + LEAD) % NBUF
                rbuf, rgsem, rssem = bufs[rb], gsem[rb], ssem[rb]

                @pl.when(r < n_chunks)
                def _refill():
                    @pl.when(r >= NBUF)
                    def _wait_store():
                        pltpu.make_async_copy(
                            rbuf, out_hbm.at[wid, 0], rssem
                        ).wait()

                    pltpu.async_copy(table_hbm.at[idx_v.at[r]], rbuf, rgsem)

                buf = bufs[b]
                pltpu.make_async_copy(
                    table_hbm.at[idx_v.at[c]], buf, gsem[b]
                ).wait()
                pltpu.async_copy(buf, out_hbm.at[wid, c], ssem[b])
            return carry

        lax.fori_loop(0, n_groups, group, 0)

        for b in range(NBUF):
            pltpu.make_async_copy(bufs[b], out_hbm.at[wid, 0], ssem[b]).wait()

    return lookup


@jax.jit
def kernel(x, table):
    num_data, seq_len = x.shape
    total = num_data * seq_len
    n_chunks = total // (NW * CHUNK)
    vocab, d = table.shape
    tpad = _transpose_scale(jnp.swapaxes(table, 0, 1))
    tflat = tpad.reshape(2 * vocab, d)
    idx = (x * 2).reshape(NW, n_chunks, CHUNK).astype(jnp.int32)
    out = _make_lookup(n_chunks)(idx, tflat)
    return out.reshape(num_data, seq_len, MODEL_DIM)


# speedup vs baseline: 1.2361x; 1.2361x over previous
import functools

import jax
import jax.numpy as jnp
from jax import lax
from jax.experimental import pallas as pl
from jax.experimental.pallas import tpu as pltpu
from jax.experimental.pallas import tpu_sc as plsc

MODEL_DIM = 64
SCALE = float(MODEL_DIM) ** 0.5

_info = plsc.get_sparse_core_info()
NC, NS, L = _info.num_cores, _info.num_subcores, _info.num_lanes
NW = NC * NS

CHUNK = 128
NBUF = 5
LEAD = 2

TBLOCK = 2048


def _transpose_scale(table_t):
    d, v = table_t.shape
    grid = pl.cdiv(v, TBLOCK)

    def body(in_ref, out_ref):
        x = in_ref[...]
        xt = jnp.swapaxes(x, 0, 1) * jnp.float32(SCALE)
        out_ref[:, :d] = xt
        out_ref[:, d:] = jnp.zeros((TBLOCK, d), jnp.float32)

    return pl.pallas_call(
        body,
        grid=(grid,),
        in_specs=[pl.BlockSpec((d, TBLOCK), lambda g: (0, g))],
        out_specs=pl.BlockSpec((TBLOCK, 2 * d), lambda g: (g, 0)),
        out_shape=jax.ShapeDtypeStruct((v, 2 * d), jnp.float32),
    )(table_t)


def _make_lookup(n_chunks):
    assert n_chunks % NBUF == 0 and n_chunks >= NBUF + LEAD
    n_groups = n_chunks // NBUF
    mesh = plsc.VectorSubcoreMesh(core_axis_name="c", subcore_axis_name="s")

    scratch = [pltpu.VMEM((n_chunks, CHUNK), jnp.int32)]
    scratch += [pltpu.VMEM((CHUNK, MODEL_DIM), jnp.float32) for _ in range(NBUF)]
    scratch += [pltpu.SemaphoreType.DMA for _ in range(2 * NBUF)]

    @functools.partial(
        pl.kernel,
        mesh=mesh,
        compiler_params=pltpu.CompilerParams(use_tc_tiling_on_sc=False),
        out_type=jax.ShapeDtypeStruct((NW, n_chunks, CHUNK, MODEL_DIM), jnp.float32),
        scratch_types=scratch,
    )
    def lookup(idx_hbm, table_hbm, out_hbm, idx_v, *bufs_and_sems):
        bufs = bufs_and_sems[:NBUF]
        gsem = bufs_and_sems[NBUF:2 * NBUF]
        ssem = bufs_and_sems[2 * NBUF:]
        wid = lax.axis_index("s") * NC + lax.axis_index("c")
        pltpu.sync_copy(idx_hbm.at[wid], idx_v)

        for c0 in range(LEAD):
            pltpu.async_copy(table_hbm.at[idx_v.at[c0]], bufs[c0], gsem[c0])

        def group(g, carry):
            for b in range(NBUF):
                c = g * NBUF + b
                r = c + LEAD
                rb = (b + LEAD) % NBUF
                rbuf, rgsem, rssem = bufs[rb], gsem[rb], ssem[rb]

                @pl.when(r < n_chunks)
                def _refill():
                    @pl.when(r >= NBUF)
                    def _wait_store():
                        pltpu.make_async_copy(
                            rbuf, out_hbm.at[wid, 0], rssem
                        ).wait()

                    pltpu.async_copy(table_hbm.at[idx_v.at[r]], rbuf, rgsem)

                buf = bufs[b]
                pltpu.make_async_copy(
                    table_hbm.at[idx_v.at[c]], buf, gsem[b]
                ).wait()
                pltpu.async_copy(buf, out_hbm.at[wid, c], ssem[b])
            return carry

        lax.fori_loop(0, n_groups, group, 0)

        for b in range(NBUF):
            pltpu.make_async_copy(bufs[b], out_hbm.at[wid, 0], ssem[b]).wait()

    return lookup


@jax.jit
def kernel(x, table):
    num_data, seq_len = x.shape
    total = num_data * seq_len
    n_chunks = total // (NW * CHUNK)
    vocab, d = table.shape
    tpad = _transpose_scale(jnp.swapaxes(table, 0, 1))
    tflat = tpad.reshape(2 * vocab, d)
    idx = (x * 2).reshape(NW, n_chunks, CHUNK).astype(jnp.int32)
    out = _make_lookup(n_chunks)(idx, tflat)
    return out.reshape(num_data, seq_len, MODEL_DIM)
